# Initial kernel scaffold; baseline (speedup 1.0000x reference)
import jax
import jax.numpy as jnp
from jax.experimental import pallas as pl


def kernel(x, adj, W, b):
    raise NotImplementedError("write your pallas kernel here")


# speedup vs baseline: 19895.1453x; 19895.1453x over previous
import jax
import jax.numpy as jnp
from jax.experimental import pallas as pl


def _gcn_body(x_ref, adj_ref, w_ref, b_ref, out_ref):
    adj = adj_ref[:]
    colsum = jnp.sum(adj, axis=0, keepdims=True)
    dinv = jax.lax.rsqrt(2.0 * colsum + 1.0)
    ht = jax.lax.dot_general(w_ref[:], x_ref[:], (((0,), (1,)), ((), ())),
                             preferred_element_type=jnp.float32)
    hht = ht * dinv
    st = jnp.dot(hht, adj, preferred_element_type=jnp.float32)
    outt = dinv * (2.0 * st + hht) + b_ref[:]
    out_ref[:] = jnp.tanh(outt)


def kernel(x, adj, W, b):
    n = x.shape[0]
    out_c = W.shape[1]
    return pl.pallas_call(
        _gcn_body,
        out_shape=jax.ShapeDtypeStruct((out_c, n), jnp.float32),
    )(x, adj, W, b.reshape(out_c, 1))
